# Initial kernel scaffold; baseline (speedup 1.0000x reference)
import jax
import jax.numpy as jnp
from jax.experimental import pallas as pl


def kernel(inputs, kernel):
    raise NotImplementedError("write your pallas kernel here")


# speedup vs baseline: 1.0615x; 1.0615x over previous
import jax
import jax.numpy as jnp
from jax import lax
from jax.experimental import pallas as pl
from jax.experimental.pallas import tpu as pltpu
from jax.experimental.pallas import tpu_sc as plsc

NUM_BUCKETS = 1000000
BATCH = 16384
LANES = 128
NC = 2
NS = 16
NW = NC * NS
ROWS = BATCH // LANES
RPW = ROWS // NW


def _gather_body(idx_hbm, tab_hbm, out_hbm, idx_v, rows_v, sem):
    wid = lax.axis_index("s") * NC + lax.axis_index("c")
    base = wid * RPW
    pltpu.sync_copy(idx_hbm.at[pl.ds(base, RPW)], idx_v)
    copies = [
        pltpu.async_copy(tab_hbm.at[idx_v.at[j]], rows_v.at[j], sem)
        for j in range(RPW)
    ]
    for cp in copies:
        cp.wait()
    pltpu.sync_copy(rows_v, out_hbm.at[pl.ds(base, RPW)])


def kernel(inputs, kernel):
    idx = inputs.reshape(ROWS, LANES)
    tab = kernel.reshape(NUM_BUCKETS)
    mesh = plsc.VectorSubcoreMesh(core_axis_name="c", subcore_axis_name="s")
    out = pl.kernel(
        _gather_body,
        mesh=mesh,
        out_type=jax.ShapeDtypeStruct((ROWS, LANES), jnp.float32),
        scratch_types=[
            pltpu.VMEM((RPW, LANES), jnp.int32),
            pltpu.VMEM((RPW, LANES), jnp.float32),
            pltpu.SemaphoreType.DMA,
        ],
    )(idx, tab)
    return out.reshape(BATCH, 1, 1)


# speedup vs baseline: 2.4084x; 2.2689x over previous
import jax
import jax.numpy as jnp
from jax import lax
from jax.experimental import pallas as pl
from jax.experimental.pallas import tpu as pltpu
from jax.experimental.pallas import tpu_sc as plsc

NUM_BUCKETS = 1000000
PADDED_BUCKETS = 1000448
BATCH = 16384
LANES = 128
NC = 2
NS = 16
NW = NC * NS
ROWS = BATCH // LANES
RPW = ROWS // NW


def _gather_body(idx_hbm, tab_hbm, out_hbm, idx_v, rows_v, idx_sem, sem):
    wid = lax.axis_index("s") * NC + lax.axis_index("c")
    base = wid * RPW
    idx_cps = [
        pltpu.async_copy(idx_hbm.at[base + j], idx_v.at[j], idx_sem)
        for j in range(RPW)
    ]
    copies = []
    for j in range(RPW):
        idx_cps[j].wait()
        copies.append(
            pltpu.async_copy(tab_hbm.at[idx_v.at[j]], rows_v.at[j], sem)
        )
    out_cps = []
    for j in range(RPW):
        copies[j].wait()
        out_cps.append(
            pltpu.async_copy(rows_v.at[j], out_hbm.at[base + j], idx_sem)
        )
    for cp in out_cps:
        cp.wait()


def kernel(inputs, kernel):
    idx = inputs.reshape(ROWS, LANES)
    tab = jnp.pad(kernel, ((0, PADDED_BUCKETS - NUM_BUCKETS), (0, 0)))
    tab = tab.reshape(PADDED_BUCKETS)
    mesh = plsc.VectorSubcoreMesh(core_axis_name="c", subcore_axis_name="s")
    out = pl.kernel(
        _gather_body,
        mesh=mesh,
        out_type=jax.ShapeDtypeStruct((ROWS, LANES), jnp.float32),
        compiler_params=pltpu.CompilerParams(
            disable_bounds_checks=True,
            disable_semaphore_checks=True,
            skip_device_barrier=True,
        ),
        scratch_types=[
            pltpu.VMEM((RPW, LANES), jnp.int32),
            pltpu.VMEM((RPW, LANES), jnp.float32),
            pltpu.SemaphoreType.DMA,
            pltpu.SemaphoreType.DMA,
        ],
    )(idx, tab)
    return out.reshape(BATCH, 1, 1)
